# baseline (speedup 1.0000x reference)
import functools
import math

import jax
import jax.numpy as jnp
from jax import lax
from jax.experimental import pallas as pl
from jax.experimental.pallas import tpu as pltpu
from jax.experimental.pallas import tpu_sc as plsc

LANES = 16
NC, NS = 2, 16
NW = NC * NS
CHUNK = 128
NBUF = 5


def _build(b_dim, l_dim, d):
    nb = b_dim // CHUNK
    n_units = l_dim * nb
    per_w = n_units // NW
    n_grp = per_w // NBUF
    scale = math.sqrt(d)
    n_cg = d // LANES

    mesh = plsc.VectorSubcoreMesh(core_axis_name="c", subcore_axis_name="s")

    @functools.partial(
        pl.kernel,
        mesh=mesh,
        compiler_params=pltpu.CompilerParams(use_tc_tiling_on_sc=False,
                                             needs_layout_passes=False),
        out_type=jax.ShapeDtypeStruct((l_dim, d // 8, nb, 8, CHUNK),
                                      jnp.float32),
        scratch_types=[
            pltpu.VMEM((per_w, CHUNK), jnp.int32),
            *[pltpu.VMEM((CHUNK, d), jnp.float32) for _ in range(NBUF)],
            *[pltpu.VMEM((d, CHUNK + 1), jnp.float32) for _ in range(NBUF)],
            *[pltpu.SemaphoreType.DMA for _ in range(2 * NBUF)],
        ],
    )
    def run(tok_hbm, table_hbm, out_hbm, idx_v, *rest):
        gbufs = rest[0:NBUF]
        sbufs = rest[NBUF:2 * NBUF]
        gsems = rest[2 * NBUF:3 * NBUF]
        ssems = rest[3 * NBUF:4 * NBUF]

        wid = lax.axis_index("s") * NC + lax.axis_index("c")
        ubase = wid * per_w

        pltpu.sync_copy(tok_hbm.at[wid], idx_v)

        iota = lax.iota(jnp.int32, LANES)
        c_vecs = [iota + cg * LANES for cg in range(n_cg)]

        def start_gather(i, b):
            pltpu.async_copy(table_hbm.at[idx_v.at[i]], gbufs[b], gsems[b])

        def wait_gather(b):
            pltpu.make_async_copy(
                table_hbm.at[pl.ds(0, CHUNK)], gbufs[b], gsems[b]).wait()

        def start_store(i, b):
            u = ubase + i
            l = u // nb
            tc = u % nb
            for tr in range(d // 8):
                pltpu.async_copy(
                    sbufs[b].at[pl.ds(tr * 8, 8), pl.ds(0, CHUNK)],
                    out_hbm.at[l, tr, tc],
                    ssems[b])

        def wait_store(b):
            for tr in range(d // 8):
                pltpu.make_async_copy(
                    sbufs[b].at[pl.ds(tr * 8, 8), pl.ds(0, CHUNK)],
                    out_hbm.at[0, 0, 0], ssems[b]).wait()

        def transform(b):
            gb, sb = gbufs[b], sbufs[b]

            @plsc.parallel_loop(0, CHUNK, step=1, unroll=8)
            def body(t):
                t_vec = iota * 0 + t
                for cg in range(n_cg):
                    val = gb[t, pl.ds(cg * LANES, LANES)] * scale
                    plsc.store_scatter(sb, [c_vecs[cg], t_vec], val)

        for b in range(NBUF):
            start_gather(b, b)

        for b in range(NBUF):
            wait_gather(b)
            transform(b)
            start_store(b, b)
            start_gather(b + NBUF, b)

        def group_body(g, _):
            for b in range(NBUF):
                i = g * NBUF + b
                wait_gather(b)
                wait_store(b)
                transform(b)
                start_store(i, b)
                start_gather(i + NBUF, b)
            return 0

        lax.fori_loop(1, n_grp - 1, group_body, 0)

        for b in range(NBUF):
            i = (n_grp - 1) * NBUF + b
            wait_gather(b)
            wait_store(b)
            transform(b)
            start_store(i, b)

        for b in range(NBUF):
            wait_store(b)

    return run


def kernel(tokens, embedding):
    b, l = tokens.shape
    vocab, d = embedding.shape
    assert b % CHUNK == 0 and (l * b // CHUNK) % (NW * NBUF) == 0
    assert d % LANES == 0 and d % 8 == 0
    tok = tokens.T.reshape(l * b // CHUNK, CHUNK)
    tok = tok.reshape(NW, l * b // (CHUNK * NW), CHUNK).astype(jnp.int32)
    out = _build(b, l, d)(tok, embedding)
    return out.transpose(2, 4, 0, 1, 3).reshape(b, l, d)


# speedup vs baseline: 1.0099x; 1.0099x over previous
import functools
import math

import jax
import jax.numpy as jnp
from jax import lax
from jax.experimental import pallas as pl
from jax.experimental.pallas import tpu as pltpu
from jax.experimental.pallas import tpu_sc as plsc

LANES = 16
NC, NS = 2, 16
NW = NC * NS
CHUNK = 128
NBUF = 4


def _build(b_dim, l_dim, d):
    nb = b_dim // CHUNK
    n_units = l_dim * nb
    per_w = n_units // NW
    n_grp = per_w // NBUF
    scale = math.sqrt(d)
    n_cg = d // LANES

    mesh = plsc.VectorSubcoreMesh(core_axis_name="c", subcore_axis_name="s")

    @functools.partial(
        pl.kernel,
        mesh=mesh,
        compiler_params=pltpu.CompilerParams(use_tc_tiling_on_sc=False,
                                             needs_layout_passes=False),
        out_type=jax.ShapeDtypeStruct((l_dim, d // 8, nb, 8, CHUNK),
                                      jnp.float32),
        scratch_types=[
            pltpu.VMEM((per_w, CHUNK), jnp.int32),
            *[pltpu.VMEM((CHUNK, d), jnp.float32) for _ in range(NBUF)],
            *[pltpu.VMEM((d, CHUNK + 1), jnp.float32) for _ in range(NBUF)],
            *[pltpu.SemaphoreType.DMA for _ in range(2 * NBUF)],
        ],
    )
    def run(tok_hbm, table_hbm, out_hbm, idx_v, *rest):
        gbufs = rest[0:NBUF]
        sbufs = rest[NBUF:2 * NBUF]
        gsems = rest[2 * NBUF:3 * NBUF]
        ssems = rest[3 * NBUF:4 * NBUF]

        wid = lax.axis_index("s") * NC + lax.axis_index("c")
        ubase = wid * per_w

        pltpu.sync_copy(tok_hbm.at[wid], idx_v)

        iota = lax.iota(jnp.int32, LANES)
        c_vecs = [iota + cg * LANES for cg in range(n_cg)]

        def start_gather(i, b):
            pltpu.async_copy(table_hbm.at[idx_v.at[i]], gbufs[b], gsems[b])

        def wait_gather(b):
            pltpu.make_async_copy(
                table_hbm.at[pl.ds(0, CHUNK)], gbufs[b], gsems[b]).wait()

        def start_store(i, b):
            u = ubase + i
            l = u // nb
            tc = u % nb
            for tr in range(d // 8):
                pltpu.async_copy(
                    sbufs[b].at[pl.ds(tr * 8, 8), pl.ds(0, CHUNK)],
                    out_hbm.at[l, tr, tc],
                    ssems[b])

        def wait_store(b):
            for tr in range(d // 8):
                pltpu.make_async_copy(
                    sbufs[b].at[pl.ds(tr * 8, 8), pl.ds(0, CHUNK)],
                    out_hbm.at[0, 0, 0], ssems[b]).wait()

        def transform(b):
            gb, sb = gbufs[b], sbufs[b]

            @plsc.parallel_loop(0, CHUNK, step=1, unroll=8)
            def body(t):
                t_vec = iota * 0 + t
                for cg in range(n_cg):
                    val = gb[t, pl.ds(cg * LANES, LANES)] * scale
                    plsc.store_scatter(sb, [c_vecs[cg], t_vec], val)

        for b in range(NBUF):
            start_gather(b, b)

        for b in range(NBUF):
            wait_gather(b)
            transform(b)
            start_store(b, b)
            start_gather(b + NBUF, b)

        def group_body(g, _):
            for b in range(NBUF):
                i = g * NBUF + b
                wait_gather(b)
                wait_store(b)
                transform(b)
                start_store(i, b)
                start_gather(i + NBUF, b)
            return 0

        lax.fori_loop(1, n_grp - 1, group_body, 0)

        for b in range(NBUF):
            i = (n_grp - 1) * NBUF + b
            wait_gather(b)
            wait_store(b)
            transform(b)
            start_store(i, b)

        for b in range(NBUF):
            wait_store(b)

    return run


def kernel(tokens, embedding):
    b, l = tokens.shape
    vocab, d = embedding.shape
    assert b % CHUNK == 0 and (l * b // CHUNK) % (NW * NBUF) == 0
    assert d % LANES == 0 and d % 8 == 0
    tok = tokens.T.reshape(l * b // CHUNK, CHUNK)
    tok = tok.reshape(NW, l * b // (CHUNK * NW), CHUNK).astype(jnp.int32)
    out = _build(b, l, d)(tok, embedding)
    return out.transpose(2, 4, 0, 1, 3).reshape(b, l, d)


# speedup vs baseline: 1.5205x; 1.5057x over previous
import functools
import math

import jax
import jax.numpy as jnp
from jax import lax
from jax.experimental import pallas as pl
from jax.experimental.layout import Layout, with_layout_constraint
from jax.experimental.pallas import tpu as pltpu
from jax.experimental.pallas import tpu_sc as plsc

LANES = 16
NC, NS = 2, 16
NW = NC * NS
CHUNK = 128
NBUF = 4


def _build(b_dim, l_dim, d):
    nb = b_dim // CHUNK
    n_units = l_dim * nb
    per_w = n_units // NW
    n_grp = per_w // NBUF
    scale = math.sqrt(d)
    n_cg = d // LANES

    mesh = plsc.VectorSubcoreMesh(core_axis_name="c", subcore_axis_name="s")

    @functools.partial(
        pl.kernel,
        mesh=mesh,
        compiler_params=pltpu.CompilerParams(use_tc_tiling_on_sc=False,
                                             needs_layout_passes=False),
        out_type=jax.ShapeDtypeStruct((l_dim, d // 8, nb, 8, CHUNK),
                                      jnp.float32),
        scratch_types=[
            pltpu.VMEM((per_w, CHUNK), jnp.int32),
            *[pltpu.VMEM((CHUNK, d), jnp.float32) for _ in range(NBUF)],
            *[pltpu.VMEM((d, CHUNK + 1), jnp.float32) for _ in range(NBUF)],
            *[pltpu.SemaphoreType.DMA for _ in range(2 * NBUF)],
        ],
    )
    def run(tok_hbm, table_hbm, out_hbm, idx_v, *rest):
        gbufs = rest[0:NBUF]
        sbufs = rest[NBUF:2 * NBUF]
        gsems = rest[2 * NBUF:3 * NBUF]
        ssems = rest[3 * NBUF:4 * NBUF]

        wid = lax.axis_index("s") * NC + lax.axis_index("c")
        ubase = wid * per_w

        pltpu.sync_copy(tok_hbm.at[wid], idx_v)

        iota = lax.iota(jnp.int32, LANES)
        c_vecs = [iota + cg * LANES for cg in range(n_cg)]

        def start_gather(i, b):
            pltpu.async_copy(table_hbm.at[idx_v.at[i]], gbufs[b], gsems[b])

        def wait_gather(b):
            pltpu.make_async_copy(
                table_hbm.at[pl.ds(0, CHUNK)], gbufs[b], gsems[b]).wait()

        def start_store(i, b):
            u = ubase + i
            l = u // nb
            tc = u % nb
            for tr in range(d // 8):
                pltpu.async_copy(
                    sbufs[b].at[pl.ds(tr * 8, 8), pl.ds(0, CHUNK)],
                    out_hbm.at[l, tr, tc],
                    ssems[b])

        def wait_store(b):
            for tr in range(d // 8):
                pltpu.make_async_copy(
                    sbufs[b].at[pl.ds(tr * 8, 8), pl.ds(0, CHUNK)],
                    out_hbm.at[0, 0, 0], ssems[b]).wait()

        def transform(b):
            gb, sb = gbufs[b], sbufs[b]

            @plsc.parallel_loop(0, CHUNK, step=1, unroll=8)
            def body(t):
                t_vec = iota * 0 + t
                for cg in range(n_cg):
                    val = gb[t, pl.ds(cg * LANES, LANES)] * scale
                    plsc.store_scatter(sb, [c_vecs[cg], t_vec], val)

        for b in range(NBUF):
            start_gather(b, b)

        for b in range(NBUF):
            wait_gather(b)
            transform(b)
            start_store(b, b)
            start_gather(b + NBUF, b)

        def group_body(g, _):
            for b in range(NBUF):
                i = g * NBUF + b
                wait_gather(b)
                wait_store(b)
                transform(b)
                start_store(i, b)
                start_gather(i + NBUF, b)
            return 0

        lax.fori_loop(1, n_grp - 1, group_body, 0)

        for b in range(NBUF):
            i = (n_grp - 1) * NBUF + b
            wait_gather(b)
            wait_store(b)
            transform(b)
            start_store(i, b)

        for b in range(NBUF):
            wait_store(b)

    return run


def kernel(tokens, embedding):
    b, l = tokens.shape
    vocab, d = embedding.shape
    assert b % CHUNK == 0 and (l * b // CHUNK) % (NW * NBUF) == 0
    assert d % LANES == 0 and d % 8 == 0
    tok = tokens.T.reshape(l * b // CHUNK, CHUNK)
    tok = tok.reshape(NW, l * b // (CHUNK * NW), CHUNK).astype(jnp.int32)
    tbl = with_layout_constraint(
        embedding, Layout(major_to_minor=(0, 1), tiling=((8,), (1024,))))
    out = _build(b, l, d)(tok, tbl)
    return out.transpose(2, 4, 0, 1, 3).reshape(b, l, d)
